# baseline (speedup 1.0000x reference)
import functools

import jax
import jax.numpy as jnp
from jax import lax
from jax.experimental import pallas as pl
from jax.experimental.pallas import tpu as pltpu
from jax.experimental.pallas import tpu_sc as plsc

DIM = 64
BATCH2 = 32768
NC, NS, LANES = 2, 16, 16
NW = NC * NS
ROWS_PER_W = BATCH2 // NW
CHUNK = 128
NCHUNK = ROWS_PER_W // CHUNK

_LN2 = 0.6931471805599453


def _softplus16(x):
    t = 1.0 + jnp.exp(x)
    i = lax.bitcast_convert_type(t, jnp.int32)
    e = lax.shift_right_arithmetic(i, 23) - 127
    m = lax.bitcast_convert_type(
        lax.bitwise_or(lax.bitwise_and(i, 0x007FFFFF), 0x3F800000),
        jnp.float32)
    z = (m - 1.0) / (m + 1.0)
    z2 = z * z
    p = z * (2.0 + z2 * (2.0 / 3.0 + z2 * (2.0 / 5.0 + z2 * (2.0 / 7.0))))
    ln_t = e.astype(jnp.float32) * _LN2 + p
    return jnp.where(x > 20.0, x, ln_t)


def _sc_body(idxh_hbm, poff_hbm, u_hbm, v_hbm, out_hbm,
             idxh_v, poff_v, bufu, bufv, bufo, gsem0, gsem1, osem0, osem1):
    wid = lax.axis_index("s") * NC + lax.axis_index("c")
    pltpu.sync_copy(idxh_hbm.at[wid], idxh_v)
    pltpu.sync_copy(poff_hbm.at[wid], poff_v)
    base = wid * ROWS_PER_W
    gsems = (gsem0, gsem1)
    osems = (osem0, osem1)

    def start_gather(c, slot):
        pltpu.async_copy(u_hbm.at[idxh_v.at[c]], bufu.at[slot], gsems[slot])
        pltpu.async_copy(v_hbm.at[idxh_v.at[c]], bufv.at[slot], gsems[slot])

    def wait_gather(slot):
        dummy = u_hbm.at[pl.ds(0, CHUNK)]
        pltpu.make_async_copy(dummy, bufu.at[slot], gsems[slot]).wait()
        pltpu.make_async_copy(dummy, bufv.at[slot], gsems[slot]).wait()

    def out_rows(c):
        return out_hbm.at[pl.ds(pl.multiple_of(base + c * CHUNK, CHUNK),
                                CHUNK)]

    def start_out(c, slot):
        pltpu.async_copy(bufo.at[slot], out_rows(c), osems[slot])

    def wait_out(slot):
        pltpu.make_async_copy(bufo.at[slot], out_rows(0), osems[slot]).wait()

    def assemble(c, slot):
        def grp(g, carry):
            pv = poff_v[c, pl.ds(g * LANES, LANES)]
            for l in range(LANES):
                r = g * LANES + l
                off = pv[l]
                for k in range(DIM // LANES):
                    sl = pl.ds(k * LANES, LANES)
                    sr = pl.ds(DIM + k * LANES, LANES)
                    sg = pl.ds(off + k * LANES, LANES)
                    bufo[slot, r, sl] = bufu[slot, r, sg]
                    bufo[slot, r, sr] = bufv[slot, r, sg]
            return carry
        lax.fori_loop(0, CHUNK // LANES, grp, 0)

        def sp(r, carry):
            for k in range(DIM // LANES):
                sr = pl.ds(DIM + k * LANES, LANES)
                bufo[slot, r, sr] = _softplus16(bufo[slot, r, sr])
            return carry
        lax.fori_loop(0, CHUNK, sp, 0)

    def step(c, slot, first=False, last=False):
        if not last:
            start_gather(c + 1, 1 - slot)
        wait_gather(slot)
        if not first:
            wait_out(slot)
        assemble(c, slot)
        start_out(c, slot)

    start_gather(0, 0)
    step(0, 0, first=True)
    step(1, 1, first=True)

    def pair(i, carry):
        c = i * 2
        step(c, 0)
        step(c + 1, 1)
        return carry

    lax.fori_loop(1, NCHUNK // 2 - 1, pair, 0)

    step(NCHUNK - 2, 0)
    step(NCHUNK - 1, 1, last=True)
    wait_out(0)
    wait_out(1)


@jax.jit
def _hard_box_sc(idx3h, poff3, u2, v2):
    mesh = plsc.VectorSubcoreMesh(core_axis_name="c", subcore_axis_name="s")
    k = functools.partial(
        pl.kernel,
        out_type=jax.ShapeDtypeStruct((BATCH2, 2 * DIM), jnp.float32),
        mesh=mesh,
        scratch_types=[
            pltpu.VMEM((NCHUNK, CHUNK), jnp.int32),
            pltpu.VMEM((NCHUNK, CHUNK), jnp.int32),
            pltpu.VMEM((2, CHUNK, 2 * DIM), jnp.float32),
            pltpu.VMEM((2, CHUNK, 2 * DIM), jnp.float32),
            pltpu.VMEM((2, CHUNK, 2 * DIM), jnp.float32),
            pltpu.SemaphoreType.DMA,
            pltpu.SemaphoreType.DMA,
            pltpu.SemaphoreType.DMA,
            pltpu.SemaphoreType.DMA,
        ],
    )(_sc_body)
    return k(idx3h, poff3, u2, v2)


def kernel(idxs, U, V):
    idx = idxs.reshape(-1).astype(jnp.int32)
    idx3h = lax.shift_right_logical(idx, 1).reshape(NW, NCHUNK, CHUNK)
    poff3 = (lax.bitwise_and(idx, 1) * DIM).reshape(NW, NCHUNK, CHUNK)
    u2 = U.reshape(U.shape[0] // 2, 2 * DIM)
    v2 = V.reshape(V.shape[0] // 2, 2 * DIM)
    out = _hard_box_sc(idx3h, poff3, u2, v2)
    return out.reshape(idxs.shape[0], 2, 2, DIM)


# speedup vs baseline: 1.0706x; 1.0706x over previous
import functools

import jax
import jax.numpy as jnp
from jax import lax
from jax.experimental import pallas as pl
from jax.experimental.pallas import tpu as pltpu
from jax.experimental.pallas import tpu_sc as plsc

DIM = 64
BATCH2 = 32768
NC, NS, LANES = 2, 16, 16
NW = NC * NS
LBL_PER_W = BATCH2 // NW
CHUNK = 128
NCHUNK = LBL_PER_W // CHUNK
OROWS = CHUNK // 2

_LN2 = 0.6931471805599453


def _softplus16(x):
    t = 1.0 + jnp.exp(x)
    i = lax.bitcast_convert_type(t, jnp.int32)
    e = lax.shift_right_arithmetic(i, 23) - 127
    m = lax.bitcast_convert_type(
        lax.bitwise_or(lax.bitwise_and(i, 0x007FFFFF), 0x3F800000),
        jnp.float32)
    z = (m - 1.0) / (m + 1.0)
    z2 = z * z
    p = z * (2.0 + z2 * (2.0 / 3.0 + z2 * (2.0 / 5.0 + z2 * (2.0 / 7.0))))
    ln_t = e.astype(jnp.float32) * _LN2 + p
    return jnp.where(x > 20.0, x, ln_t)


def _make_body(softplus):
    def _sc_body(idxh_hbm, poff_hbm, tbl_hbm, out_hbm,
                 idxh_v, poff_v, bufg, bufo, gsem0, gsem1, osem0, osem1):
        wid = lax.axis_index("s") * NC + lax.axis_index("c")
        pltpu.sync_copy(idxh_hbm.at[wid], idxh_v)
        pltpu.sync_copy(poff_hbm.at[wid], poff_v)
        obase = wid * (LBL_PER_W // 2)
        gsems = (gsem0, gsem1)
        osems = (osem0, osem1)

        def start_gather(c, slot):
            pltpu.async_copy(tbl_hbm.at[idxh_v.at[c]], bufg.at[slot],
                             gsems[slot])

        def wait_gather(slot):
            dummy = tbl_hbm.at[pl.ds(0, CHUNK)]
            pltpu.make_async_copy(dummy, bufg.at[slot], gsems[slot]).wait()

        def out_rows(c):
            return out_hbm.at[pl.ds(pl.multiple_of(obase + c * OROWS, OROWS),
                                    OROWS)]

        def start_out(c, slot):
            pltpu.async_copy(bufo.at[slot], out_rows(c), osems[slot])

        def wait_out(slot):
            pltpu.make_async_copy(bufo.at[slot], out_rows(0),
                                  osems[slot]).wait()

        def assemble(c, slot):
            def grp(g, carry):
                pv = poff_v[c, pl.ds(g * LANES, LANES)]
                for l in range(LANES):
                    j = g * LANES + l
                    off = pv[l]
                    half = (l & 1) * DIM
                    for k in range(DIM // LANES):
                        sd = pl.ds(half + k * LANES, LANES)
                        sg = pl.ds(off + k * LANES, LANES)
                        bufo[slot, g * (LANES // 2) + l // 2, sd] = \
                            bufg[slot, j, sg]
                return carry
            lax.fori_loop(0, CHUNK // LANES, grp, 0)

            if softplus:
                def sp(r, carry):
                    for k in range(2 * DIM // LANES):
                        s = pl.ds(k * LANES, LANES)
                        bufo[slot, r, s] = _softplus16(bufo[slot, r, s])
                    return carry
                lax.fori_loop(0, OROWS, sp, 0)

        def step(c, slot, first=False, last=False):
            if not last:
                start_gather(c + 1, 1 - slot)
            wait_gather(slot)
            if not first:
                wait_out(slot)
            assemble(c, slot)
            start_out(c, slot)

        start_gather(0, 0)
        step(0, 0, first=True)
        step(1, 1, first=True)

        def pair(i, carry):
            c = i * 2
            step(c, 0)
            step(c + 1, 1)
            return carry

        lax.fori_loop(1, NCHUNK // 2 - 1, pair, 0)

        step(NCHUNK - 2, 0)
        step(NCHUNK - 1, 1, last=True)
        wait_out(0)
        wait_out(1)

    return _sc_body


def _gather_call(softplus):
    mesh = plsc.VectorSubcoreMesh(core_axis_name="c", subcore_axis_name="s")
    return functools.partial(
        pl.kernel,
        out_type=jax.ShapeDtypeStruct((BATCH2 // 2, 2 * DIM), jnp.float32),
        mesh=mesh,
        scratch_types=[
            pltpu.VMEM((NCHUNK, CHUNK), jnp.int32),
            pltpu.VMEM((NCHUNK, CHUNK), jnp.int32),
            pltpu.VMEM((2, CHUNK, 2 * DIM), jnp.float32),
            pltpu.VMEM((2, OROWS, 2 * DIM), jnp.float32),
            pltpu.SemaphoreType.DMA,
            pltpu.SemaphoreType.DMA,
            pltpu.SemaphoreType.DMA,
            pltpu.SemaphoreType.DMA,
        ],
    )(_make_body(softplus))


@jax.jit
def _hard_box_sc(idx3h, poff3, u2, v2):
    mins2 = _gather_call(False)(idx3h, poff3, u2)
    deltas2 = _gather_call(True)(idx3h, poff3, v2)
    return mins2, deltas2


def kernel(idxs, U, V):
    idx = idxs.reshape(-1).astype(jnp.int32)
    idx3h = lax.shift_right_logical(idx, 1).reshape(NW, NCHUNK, CHUNK)
    poff3 = (lax.bitwise_and(idx, 1) * DIM).reshape(NW, NCHUNK, CHUNK)
    u2 = U.reshape(U.shape[0] // 2, 2 * DIM)
    v2 = V.reshape(V.shape[0] // 2, 2 * DIM)
    mins2, deltas2 = _hard_box_sc(idx3h, poff3, u2, v2)
    mins = mins2.reshape(idxs.shape[0], 2, DIM)
    deltas = deltas2.reshape(idxs.shape[0], 2, DIM)
    return jnp.stack([mins, deltas], axis=-2)
